# baseline (speedup 1.0000x reference)
import jax
import jax.numpy as jnp
from jax.experimental import pallas as pl
from jax.experimental.pallas import tpu as pltpu

BLK = 128
NB = 16
KEEP = 2
NEG = -1e9
FMIN = -3.0e38


def _fused_kernel(q_ref, k_ref, v_ref, o_ref, idx_vmem, idx_smem, sem):
    q = q_ref[0, 0]
    k = k_ref[0, 0]
    S, D = q.shape
    scale = jnp.float32(1.0) / jnp.sqrt(jnp.float32(D))

    qp = jnp.concatenate(
        [jnp.sum(q[i * BLK:(i + 1) * BLK, :], axis=0, keepdims=True)
         for i in range(NB)], axis=0) * jnp.float32(1.0 / BLK)
    kp = jnp.concatenate(
        [jnp.sum(k[i * BLK:(i + 1) * BLK, :], axis=0, keepdims=True)
         for i in range(NB)], axis=0) * jnp.float32(1.0 / BLK)
    s = jax.lax.dot_general(qp.astype(jnp.bfloat16), kp.astype(jnp.bfloat16),
                            (((1,), (1,)), ((), ())),
                            preferred_element_type=jnp.float32) * scale
    m = jnp.max(s, axis=1, keepdims=True)
    e = jnp.exp(s - m)
    p = e / jnp.sum(e, axis=1, keepdims=True)
    col = jax.lax.broadcasted_iota(jnp.int32, (NB, NB), 1)
    m1 = jnp.max(p, axis=1, keepdims=True)
    a1 = jnp.min(jnp.where(p >= m1, col, NB), axis=1)
    p2 = jnp.where(col == a1[:, None], FMIN, p)
    m2 = jnp.max(p2, axis=1, keepdims=True)
    a2 = jnp.min(jnp.where(p2 >= m2, col, NB), axis=1)
    idx_vmem[...] = jnp.stack([a1, a2], axis=0)
    copy = pltpu.make_async_copy(idx_vmem, idx_smem, sem)
    copy.start()
    copy.wait()

    scale_a = jnp.float32(0.125)
    for qb in range(NB):
        i0 = idx_smem[0, qb]
        i1 = idx_smem[1, qb]
        qblk = q_ref[0, 0, qb * BLK:(qb + 1) * BLK, :]
        kc = jnp.concatenate(
            [k_ref[0, 0, pl.ds(i0 * BLK, BLK), :],
             k_ref[0, 0, pl.ds(i1 * BLK, BLK), :],
             k_ref[0, 0, qb * BLK:(qb + 1) * BLK, :]], axis=0)
        vc = jnp.concatenate(
            [v_ref[0, 0, pl.ds(i0 * BLK, BLK), :],
             v_ref[0, 0, pl.ds(i1 * BLK, BLK), :],
             v_ref[0, 0, qb * BLK:(qb + 1) * BLK, :]], axis=0)
        sc = jnp.dot(qblk, kc.T, preferred_element_type=jnp.float32) * scale_a
        dup = jnp.logical_or(i0 == qb, i1 == qb)
        colmask = jax.lax.broadcasted_iota(jnp.int32, (1, 3 * BLK), 1) >= 2 * BLK
        sc = jnp.where(jnp.logical_and(dup, colmask), NEG, sc)
        mx = jnp.max(sc, axis=1, keepdims=True)
        pr = jnp.exp(sc - mx)
        denom = jnp.sum(pr, axis=1, keepdims=True)
        acc = jnp.dot(pr, vc, preferred_element_type=jnp.float32)
        o_ref[0, 0, qb * BLK:(qb + 1) * BLK, :] = acc / denom


def kernel(q, k, v):
    B, H, S, D = q.shape
    return pl.pallas_call(
        _fused_kernel,
        grid=(B, H),
        in_specs=[
            pl.BlockSpec((1, 1, S, D), lambda b, h: (b, h, 0, 0)),
            pl.BlockSpec((1, 1, S, D), lambda b, h: (b, h, 0, 0)),
            pl.BlockSpec((1, 1, S, D), lambda b, h: (b, h, 0, 0)),
        ],
        out_specs=pl.BlockSpec((1, 1, S, D), lambda b, h: (b, h, 0, 0)),
        out_shape=jax.ShapeDtypeStruct((B, H, S, D), jnp.float32),
        scratch_shapes=[
            pltpu.VMEM((2, NB), jnp.int32),
            pltpu.SMEM((2, NB), jnp.int32),
            pltpu.SemaphoreType.DMA,
        ],
    )(q, k, v)


# speedup vs baseline: 1.1016x; 1.1016x over previous
import jax
import jax.numpy as jnp
from jax.experimental import pallas as pl
from jax.experimental.pallas import tpu as pltpu

BLK = 128
NB = 16
KEEP = 2
HPP = 4
NEG = -1e9
FMIN = -3.0e38


def _mask_rows(q, k):
    S, D = q.shape
    scale = jnp.float32(1.0) / jnp.sqrt(jnp.float32(D))
    qp = jnp.concatenate(
        [jnp.sum(q[i * BLK:(i + 1) * BLK, :], axis=0, keepdims=True)
         for i in range(NB)], axis=0) * jnp.float32(1.0 / BLK)
    kp = jnp.concatenate(
        [jnp.sum(k[i * BLK:(i + 1) * BLK, :], axis=0, keepdims=True)
         for i in range(NB)], axis=0) * jnp.float32(1.0 / BLK)
    s = jax.lax.dot_general(qp.astype(jnp.bfloat16), kp.astype(jnp.bfloat16),
                            (((1,), (1,)), ((), ())),
                            preferred_element_type=jnp.float32) * scale
    m = jnp.max(s, axis=1, keepdims=True)
    e = jnp.exp(s - m)
    p = e / jnp.sum(e, axis=1, keepdims=True)
    col = jax.lax.broadcasted_iota(jnp.int32, (NB, NB), 1)
    m1 = jnp.max(p, axis=1, keepdims=True)
    a1 = jnp.min(jnp.where(p >= m1, col, NB), axis=1)
    p2 = jnp.where(col == a1[:, None], FMIN, p)
    m2 = jnp.max(p2, axis=1, keepdims=True)
    a2 = jnp.min(jnp.where(p2 >= m2, col, NB), axis=1)
    return jnp.stack([a1, a2], axis=0)


def _fused_kernel(q_ref, k_ref, v_ref, o_ref, idx_vmem, idx_smem, sem):
    idx_vmem[...] = jnp.concatenate(
        [_mask_rows(q_ref[0, hh], k_ref[0, hh]) for hh in range(HPP)],
        axis=0)
    copy = pltpu.make_async_copy(idx_vmem, idx_smem, sem)
    copy.start()
    copy.wait()

    scale_a = jnp.float32(0.125)
    for hh in range(HPP):
        for qb in range(NB):
            i0 = idx_smem[2 * hh, qb]
            i1 = idx_smem[2 * hh + 1, qb]
            qblk = q_ref[0, hh, qb * BLK:(qb + 1) * BLK, :]
            kc = jnp.concatenate(
                [k_ref[0, hh, pl.ds(i0 * BLK, BLK), :],
                 k_ref[0, hh, pl.ds(i1 * BLK, BLK), :],
                 k_ref[0, hh, qb * BLK:(qb + 1) * BLK, :]], axis=0)
            vc = jnp.concatenate(
                [v_ref[0, hh, pl.ds(i0 * BLK, BLK), :],
                 v_ref[0, hh, pl.ds(i1 * BLK, BLK), :],
                 v_ref[0, hh, qb * BLK:(qb + 1) * BLK, :]], axis=0)
            sc = jnp.dot(qblk, kc.T,
                         preferred_element_type=jnp.float32) * scale_a
            dup = jnp.logical_or(i0 == qb, i1 == qb)
            colmask = (jax.lax.broadcasted_iota(jnp.int32, (1, 3 * BLK), 1)
                       >= 2 * BLK)
            sc = jnp.where(jnp.logical_and(dup, colmask), NEG, sc)
            mx = jnp.max(sc, axis=1, keepdims=True)
            pr = jnp.exp(sc - mx)
            denom = jnp.sum(pr, axis=1, keepdims=True)
            acc = jnp.dot(pr, vc, preferred_element_type=jnp.float32)
            o_ref[0, hh, qb * BLK:(qb + 1) * BLK, :] = acc / denom


def kernel(q, k, v):
    B, H, S, D = q.shape
    return pl.pallas_call(
        _fused_kernel,
        grid=(B, H // HPP),
        in_specs=[
            pl.BlockSpec((1, HPP, S, D), lambda b, g: (b, g, 0, 0)),
            pl.BlockSpec((1, HPP, S, D), lambda b, g: (b, g, 0, 0)),
            pl.BlockSpec((1, HPP, S, D), lambda b, g: (b, g, 0, 0)),
        ],
        out_specs=pl.BlockSpec((1, HPP, S, D), lambda b, g: (b, g, 0, 0)),
        out_shape=jax.ShapeDtypeStruct((B, H, S, D), jnp.float32),
        scratch_shapes=[
            pltpu.VMEM((2 * HPP, NB), jnp.int32),
            pltpu.SMEM((2 * HPP, NB), jnp.int32),
            pltpu.SemaphoreType.DMA,
        ],
    )(q, k, v)
